# baseline (speedup 1.0000x reference)
import functools

import jax
import jax.numpy as jnp
from jax import lax
from jax.experimental import pallas as pl
from jax.experimental.pallas import tpu as pltpu
from jax.experimental.pallas import tpu_sc as plsc

N = 10000
D = 256
H = 256
C = 16
E = 100000
NREL = 3

NC = 2
NS = 16
K = 125
NCH = E // (NS * K)
NPAD = 10240
RPT = NPAD // NS
HD = H // NC

_mesh = plsc.VectorSubcoreMesh(core_axis_name="c", subcore_axis_name="s")


DW = 128


@functools.partial(
    pl.kernel,
    out_type=jax.ShapeDtypeStruct((6, NPAD, DW), jnp.float32),
    mesh=_mesh,
    scratch_types=[
        pltpu.VMEM((K, DW), jnp.float32),
        pltpu.VMEM((NCH, K), jnp.int32),
        pltpu.MemorySpace.VMEM_SHARED((NPAD, DW), jnp.float32),
        pltpu.SemaphoreType.DMA,
    ],
)
def _deg_kernel(idx_hbm, ones_hbm, zeros_hbm, out_hbm, ones_v, idx_v, acc, ssem):
    c = lax.axis_index("c")
    s = lax.axis_index("s")
    pltpu.sync_copy(ones_hbm, ones_v)

    def _wait_s():
        pltpu.make_async_copy(ones_v, acc.at[idx_v.at[0]], ssem).wait()

    for a in range(3):
        g = 3 * c + a
        pltpu.sync_copy(zeros_hbm.at[pl.ds(s * RPT, RPT)],
                        acc.at[pl.ds(s * RPT, RPT)])
        pltpu.sync_copy(idx_hbm.at[g, s], idx_v)
        plsc.subcore_barrier()

        @pl.loop(0, NCH)
        def _(j):
            pltpu.async_copy(ones_v, acc.at[idx_v.at[j]], ssem, add=True)

            @pl.when(j >= 3)
            def _():
                _wait_s()

        for _ in range(3):
            _wait_s()
        plsc.subcore_barrier()
        pltpu.sync_copy(acc.at[pl.ds(s * RPT, RPT)],
                        out_hbm.at[g, pl.ds(s * RPT, RPT)])
        plsc.subcore_barrier()


@functools.partial(
    pl.kernel,
    out_type=jax.ShapeDtypeStruct((NREL, NC, NPAD, HD), jnp.float32),
    mesh=_mesh,
    scratch_types=[
        pltpu.VMEM((NCH, K), jnp.int32),
        pltpu.VMEM((NCH, K), jnp.int32),
        pltpu.VMEM((K, HD), jnp.float32),
        pltpu.VMEM((K, HD), jnp.float32),
        pltpu.MemorySpace.VMEM_SHARED((NPAD, HD), jnp.float32),
        pltpu.SemaphoreType.DMA,
        pltpu.SemaphoreType.DMA,
        pltpu.SemaphoreType.DMA,
        pltpu.SemaphoreType.DMA,
    ],
)
def _agg_kernel(xt_hbm, sidx_hbm, didx_hbm, zeros_hbm, out_hbm,
                src_v, dst_v, buf0, buf1, acc, gs0, gs1, ss0, ss1):
    c = lax.axis_index("c")
    s = lax.axis_index("s")
    for r in range(NREL):
        pltpu.sync_copy(zeros_hbm.at[pl.ds(s * RPT, RPT)],
                        acc.at[pl.ds(s * RPT, RPT)])
        pltpu.sync_copy(sidx_hbm.at[r, s], src_v)
        pltpu.sync_copy(didx_hbm.at[r, s], dst_v)
        plsc.subcore_barrier()

        table = xt_hbm.at[r, c]

        def _wait_g(buf, sem):
            pltpu.make_async_copy(table.at[src_v.at[0]], buf, sem).wait()

        def _wait_s(buf, sem):
            pltpu.make_async_copy(buf, acc.at[dst_v.at[0]], sem).wait()

        pltpu.async_copy(table.at[src_v.at[0]], buf0, gs0)
        pltpu.async_copy(table.at[src_v.at[1]], buf1, gs1)

        @pl.loop(0, NCH // 2)
        def _(i):
            j0 = 2 * i
            j1 = 2 * i + 1
            _wait_g(buf0, gs0)
            pltpu.async_copy(buf0, acc.at[dst_v.at[j0]], ss0, add=True)
            _wait_g(buf1, gs1)
            pltpu.async_copy(buf1, acc.at[dst_v.at[j1]], ss1, add=True)

            @pl.when(j0 + 2 < NCH)
            def _():
                _wait_s(buf0, ss0)
                pltpu.async_copy(table.at[src_v.at[j0 + 2]], buf0, gs0)

            @pl.when(j1 + 2 < NCH)
            def _():
                _wait_s(buf1, ss1)
                pltpu.async_copy(table.at[src_v.at[j1 + 2]], buf1, gs1)

        _wait_s(buf0, ss0)
        _wait_s(buf1, ss1)

        plsc.subcore_barrier()
        pltpu.sync_copy(acc.at[pl.ds(s * RPT, RPT)],
                        out_hbm.at[r, c, pl.ds(s * RPT, RPT)])
        plsc.subcore_barrier()


_BR = 1024
_NB = NPAD // _BR


def _dinv(cnt, g):
    return lax.rsqrt(jnp.maximum(cnt[g], 1.0))


def _prep_body(x_ref, cnt_ref, xt_ref):
    cnt = cnt_ref[...][:, :, 0]
    xb = x_ref[...]
    outs = []
    for r in range(NREL):
        xs = xb * _dinv(cnt, 2 * r)[:, None]
        outs.append(jnp.stack([xs[:, :HD], xs[:, HD:]], axis=0))
    xt_ref[...] = jnp.stack(outs, axis=0)


def _dense1_body(agg_ref, cnt_ref, w_ref, b_ref, xt_ref):
    cnt = cnt_ref[...][:, :, 0]
    agg = agg_ref[...]
    acc = jnp.zeros((_BR, H), jnp.float32)
    for r in range(NREL):
        a = jnp.concatenate([agg[r, 0], agg[r, 1]], axis=1)
        a = a * _dinv(cnt, 2 * r + 1)[:, None]
        acc = acc + jnp.dot(a, w_ref[...][r], preferred_element_type=jnp.float32)
    h = jnp.maximum(acc + jnp.sum(b_ref[...], axis=0)[None, :], 0.0)
    outs = []
    for r in range(NREL):
        hs = h * _dinv(cnt, 2 * r)[:, None]
        outs.append(jnp.stack([hs[:, :HD], hs[:, HD:]], axis=0))
    xt_ref[...] = jnp.stack(outs, axis=0)


def _dense2_body(agg_ref, cnt_ref, w_ref, b_ref, wc_ref, bc_ref, out_ref, acc_ref):
    cnt = cnt_ref[...][:, :, 0]
    agg = agg_ref[...]
    acc = jnp.zeros((_BR, H), jnp.float32)
    for r in range(NREL):
        a = jnp.concatenate([agg[r, 0], agg[r, 1]], axis=1)
        a = a * _dinv(cnt, 2 * r + 1)[:, None]
        acc = acc + jnp.dot(a, w_ref[...][r], preferred_element_type=jnp.float32)
    h2 = jnp.maximum(acc + jnp.sum(b_ref[...], axis=0)[None, :], 0.0)
    row = pl.program_id(0) * _BR + lax.broadcasted_iota(jnp.int32, (_BR, 1), 0)
    h2 = jnp.where(row < N, h2, 0.0)
    part = jnp.dot(jnp.ones((8, _BR), jnp.float32), h2,
                   preferred_element_type=jnp.float32)

    @pl.when(pl.program_id(0) == 0)
    def _():
        acc_ref[...] = jnp.zeros((8, H), jnp.float32)

    acc_ref[...] += part
    hg = acc_ref[0:1, :] * (1.0 / N)
    out_ref[...] = jnp.dot(hg, wc_ref[...], preferred_element_type=jnp.float32) \
        + bc_ref[0:1, :]


def _cnt_spec():
    return pl.BlockSpec((6, _BR, DW), lambda b: (0, b, 0))


def _agg_spec():
    return pl.BlockSpec((NREL, NC, _BR, HD), lambda b: (0, 0, b, 0))


def _xt_spec():
    return pl.BlockSpec((NREL, NC, _BR, HD), lambda b: (0, 0, b, 0))


_prep_call = pl.pallas_call(
    _prep_body,
    grid=(_NB,),
    in_specs=[pl.BlockSpec((_BR, D), lambda b: (b, 0)), _cnt_spec()],
    out_specs=_xt_spec(),
    out_shape=jax.ShapeDtypeStruct((NREL, NC, N, HD), jnp.float32),
)

_dense1_call = pl.pallas_call(
    _dense1_body,
    grid=(_NB,),
    in_specs=[
        _agg_spec(),
        _cnt_spec(),
        pl.BlockSpec((NREL, H, H), lambda b: (0, 0, 0)),
        pl.BlockSpec((8, H), lambda b: (0, 0)),
    ],
    out_specs=_xt_spec(),
    out_shape=jax.ShapeDtypeStruct((NREL, NC, N, HD), jnp.float32),
)

_dense2_call = pl.pallas_call(
    _dense2_body,
    grid=(_NB,),
    in_specs=[
        _agg_spec(),
        _cnt_spec(),
        pl.BlockSpec((NREL, H, H), lambda b: (0, 0, 0)),
        pl.BlockSpec((8, H), lambda b: (0, 0)),
        pl.BlockSpec((H, C), lambda b: (0, 0)),
        pl.BlockSpec((8, C), lambda b: (0, 0)),
    ],
    out_specs=pl.BlockSpec((1, C), lambda b: (0, 0)),
    out_shape=jax.ShapeDtypeStruct((1, C), jnp.float32),
    scratch_shapes=[pltpu.VMEM((8, H), jnp.float32)],
)


def _pad8(*rows):
    z = jnp.zeros((8, rows[0].shape[0]), jnp.float32)
    for i, r in enumerate(rows):
        z = z.at[i].set(r)
    return z


def kernel(x, e0, e1, e2, W1_0, b1_0, W1_1, b1_1, W1_2, b1_2,
           W2_0, b2_0, W2_1, b2_1, W2_2, b2_2, Wc, bc):
    src = jnp.stack([e0[0], e1[0], e2[0]]).astype(jnp.int32)
    dst = jnp.stack([e0[1], e1[1], e2[1]]).astype(jnp.int32)
    sidx = src.reshape(NREL, NS, NCH, K)
    didx = dst.reshape(NREL, NS, NCH, K)
    idx6 = jnp.stack([src, dst], axis=1).reshape(6, NS, NCH, K)

    zeros128 = jnp.zeros((NPAD, HD), jnp.float32)
    onesd = jnp.ones((K, DW), jnp.float32)
    zerosd = jnp.zeros((NPAD, DW), jnp.float32)

    w1 = jnp.stack([W1_0, W1_1, W1_2])
    w2 = jnp.stack([W2_0, W2_1, W2_2])
    b1 = _pad8(b1_0, b1_1, b1_2)
    b2 = _pad8(b2_0, b2_1, b2_2)
    bc8 = _pad8(bc)

    counts = _deg_kernel(idx6, onesd, zerosd)
    xt1 = _prep_call(x, counts)
    agg1 = _agg_kernel(xt1, sidx, didx, zeros128)
    xt2 = _dense1_call(agg1, counts, w1, b1)
    agg2 = _agg_kernel(xt2, sidx, didx, zeros128)
    out = _dense2_call(agg2, counts, w2, b2, Wc, bc8)
    return out


# speedup vs baseline: 1.1926x; 1.1926x over previous
import functools

import jax
import jax.numpy as jnp
from jax import lax
from jax.experimental import pallas as pl
from jax.experimental.pallas import tpu as pltpu
from jax.experimental.pallas import tpu_sc as plsc

N = 10000
D = 256
H = 256
C = 16
E = 100000
NREL = 3

NC = 2
NS = 16
K = 125
NCH = E // (NS * K)
NPAD = 10240
RPT = NPAD // NS
HD = H // NC

_mesh = plsc.VectorSubcoreMesh(core_axis_name="c", subcore_axis_name="s")


DW = 128


@functools.partial(
    pl.kernel,
    out_type=jax.ShapeDtypeStruct((6, NPAD, DW), jnp.float32),
    mesh=_mesh,
    scratch_types=[
        pltpu.VMEM((K, DW), jnp.float32),
        pltpu.VMEM((NCH, K), jnp.int32),
        pltpu.MemorySpace.VMEM_SHARED((NPAD, DW), jnp.float32),
        pltpu.SemaphoreType.DMA,
    ],
)
def _deg_kernel(idx_hbm, ones_hbm, zeros_hbm, out_hbm, ones_v, idx_v, acc, ssem):
    c = lax.axis_index("c")
    s = lax.axis_index("s")
    pltpu.sync_copy(ones_hbm, ones_v)

    def _wait_s():
        pltpu.make_async_copy(ones_v, acc.at[idx_v.at[0]], ssem).wait()

    for a in range(3):
        g = 3 * c + a
        pltpu.sync_copy(zeros_hbm.at[pl.ds(s * RPT, RPT)],
                        acc.at[pl.ds(s * RPT, RPT)])
        pltpu.sync_copy(idx_hbm.at[g, s], idx_v)
        plsc.subcore_barrier()

        @pl.loop(0, NCH)
        def _(j):
            pltpu.async_copy(ones_v, acc.at[idx_v.at[j]], ssem, add=True)

            @pl.when(j >= 3)
            def _():
                _wait_s()

        for _ in range(3):
            _wait_s()
        plsc.subcore_barrier()
        pltpu.sync_copy(acc.at[pl.ds(s * RPT, RPT)],
                        out_hbm.at[g, pl.ds(s * RPT, RPT)])
        plsc.subcore_barrier()


@functools.partial(
    pl.kernel,
    out_type=jax.ShapeDtypeStruct((NREL, NC, NPAD, HD), jnp.float32),
    mesh=_mesh,
    scratch_types=[
        pltpu.VMEM((NCH, K), jnp.int32),
        pltpu.VMEM((NCH, K), jnp.int32),
        pltpu.VMEM((K, HD), jnp.float32),
        pltpu.VMEM((K, HD), jnp.float32),
        pltpu.MemorySpace.VMEM_SHARED((NPAD, HD), jnp.float32),
        pltpu.SemaphoreType.DMA,
        pltpu.SemaphoreType.DMA,
        pltpu.SemaphoreType.DMA,
        pltpu.SemaphoreType.DMA,
    ],
)
def _agg_kernel(xt_hbm, sidx_hbm, didx_hbm, zeros_hbm, out_hbm,
                src_v, dst_v, buf0, buf1, acc, gs0, gs1, ss0, ss1):
    c = lax.axis_index("c")
    s = lax.axis_index("s")
    for r in range(NREL):
        pltpu.sync_copy(zeros_hbm.at[pl.ds(s * RPT, RPT)],
                        acc.at[pl.ds(s * RPT, RPT)])
        pltpu.sync_copy(sidx_hbm.at[r, s], src_v)
        pltpu.sync_copy(didx_hbm.at[r, s], dst_v)
        plsc.subcore_barrier()

        table = xt_hbm.at[r, c]
        pltpu.async_copy(table.at[src_v.at[0]], buf0, gs0)

        @pl.loop(0, NCH // 2)
        def _(i):
            j0 = 2 * i
            j1 = 2 * i + 1
            pltpu.async_copy(table.at[src_v.at[j1]], buf1, gs1)
            pltpu.make_async_copy(table.at[src_v.at[0]], buf0, gs0).wait()
            pltpu.sync_copy(buf0, acc.at[dst_v.at[j0]], add=True)

            @pl.when(j1 + 1 < NCH)
            def _():
                pltpu.async_copy(table.at[src_v.at[j1 + 1]], buf0, gs0)

            pltpu.make_async_copy(table.at[src_v.at[0]], buf1, gs1).wait()
            pltpu.sync_copy(buf1, acc.at[dst_v.at[j1]], add=True)

        plsc.subcore_barrier()
        pltpu.sync_copy(acc.at[pl.ds(s * RPT, RPT)],
                        out_hbm.at[r, c, pl.ds(s * RPT, RPT)])
        plsc.subcore_barrier()


_BR = 1024
_NB = NPAD // _BR


def _dinv(cnt, g):
    return lax.rsqrt(jnp.maximum(cnt[g], 1.0))


def _prep_body(x_ref, cnt_ref, xt_ref):
    cnt = cnt_ref[...][:, :, 0]
    xb = x_ref[...]
    outs = []
    for r in range(NREL):
        xs = xb * _dinv(cnt, 2 * r)[:, None]
        outs.append(jnp.stack([xs[:, :HD], xs[:, HD:]], axis=0))
    xt_ref[...] = jnp.stack(outs, axis=0)


def _dense1_body(agg_ref, cnt_ref, w_ref, b_ref, xt_ref):
    cnt = cnt_ref[...][:, :, 0]
    agg = agg_ref[...]
    acc = jnp.zeros((_BR, H), jnp.float32)
    for r in range(NREL):
        a = jnp.concatenate([agg[r, 0], agg[r, 1]], axis=1)
        a = a * _dinv(cnt, 2 * r + 1)[:, None]
        acc = acc + jnp.dot(a, w_ref[...][r], preferred_element_type=jnp.float32)
    h = jnp.maximum(acc + jnp.sum(b_ref[...], axis=0)[None, :], 0.0)
    outs = []
    for r in range(NREL):
        hs = h * _dinv(cnt, 2 * r)[:, None]
        outs.append(jnp.stack([hs[:, :HD], hs[:, HD:]], axis=0))
    xt_ref[...] = jnp.stack(outs, axis=0)


def _dense2_body(agg_ref, cnt_ref, w_ref, b_ref, wc_ref, bc_ref, out_ref, acc_ref):
    cnt = cnt_ref[...][:, :, 0]
    agg = agg_ref[...]
    acc = jnp.zeros((_BR, H), jnp.float32)
    for r in range(NREL):
        a = jnp.concatenate([agg[r, 0], agg[r, 1]], axis=1)
        a = a * _dinv(cnt, 2 * r + 1)[:, None]
        acc = acc + jnp.dot(a, w_ref[...][r], preferred_element_type=jnp.float32)
    h2 = jnp.maximum(acc + jnp.sum(b_ref[...], axis=0)[None, :], 0.0)
    row = pl.program_id(0) * _BR + lax.broadcasted_iota(jnp.int32, (_BR, 1), 0)
    h2 = jnp.where(row < N, h2, 0.0)
    part = jnp.dot(jnp.ones((8, _BR), jnp.float32), h2,
                   preferred_element_type=jnp.float32)

    @pl.when(pl.program_id(0) == 0)
    def _():
        acc_ref[...] = jnp.zeros((8, H), jnp.float32)

    acc_ref[...] += part
    hg = acc_ref[0:1, :] * (1.0 / N)
    out_ref[...] = jnp.dot(hg, wc_ref[...], preferred_element_type=jnp.float32) \
        + bc_ref[0:1, :]


def _cnt_spec():
    return pl.BlockSpec((6, _BR, DW), lambda b: (0, b, 0))


def _agg_spec():
    return pl.BlockSpec((NREL, NC, _BR, HD), lambda b: (0, 0, b, 0))


def _xt_spec():
    return pl.BlockSpec((NREL, NC, _BR, HD), lambda b: (0, 0, b, 0))


_prep_call = pl.pallas_call(
    _prep_body,
    grid=(_NB,),
    in_specs=[pl.BlockSpec((_BR, D), lambda b: (b, 0)), _cnt_spec()],
    out_specs=_xt_spec(),
    out_shape=jax.ShapeDtypeStruct((NREL, NC, N, HD), jnp.float32),
)

_dense1_call = pl.pallas_call(
    _dense1_body,
    grid=(_NB,),
    in_specs=[
        _agg_spec(),
        _cnt_spec(),
        pl.BlockSpec((NREL, H, H), lambda b: (0, 0, 0)),
        pl.BlockSpec((8, H), lambda b: (0, 0)),
    ],
    out_specs=_xt_spec(),
    out_shape=jax.ShapeDtypeStruct((NREL, NC, N, HD), jnp.float32),
)

_dense2_call = pl.pallas_call(
    _dense2_body,
    grid=(_NB,),
    in_specs=[
        _agg_spec(),
        _cnt_spec(),
        pl.BlockSpec((NREL, H, H), lambda b: (0, 0, 0)),
        pl.BlockSpec((8, H), lambda b: (0, 0)),
        pl.BlockSpec((H, C), lambda b: (0, 0)),
        pl.BlockSpec((8, C), lambda b: (0, 0)),
    ],
    out_specs=pl.BlockSpec((1, C), lambda b: (0, 0)),
    out_shape=jax.ShapeDtypeStruct((1, C), jnp.float32),
    scratch_shapes=[pltpu.VMEM((8, H), jnp.float32)],
)


def _pad8(*rows):
    z = jnp.zeros((8, rows[0].shape[0]), jnp.float32)
    for i, r in enumerate(rows):
        z = z.at[i].set(r)
    return z


def kernel(x, e0, e1, e2, W1_0, b1_0, W1_1, b1_1, W1_2, b1_2,
           W2_0, b2_0, W2_1, b2_1, W2_2, b2_2, Wc, bc):
    src = jnp.stack([e0[0], e1[0], e2[0]]).astype(jnp.int32)
    dst = jnp.stack([e0[1], e1[1], e2[1]]).astype(jnp.int32)
    sidx = src.reshape(NREL, NS, NCH, K)
    didx = dst.reshape(NREL, NS, NCH, K)
    idx6 = jnp.stack([src, dst], axis=1).reshape(6, NS, NCH, K)

    zeros128 = jnp.zeros((NPAD, HD), jnp.float32)
    onesd = jnp.ones((K, DW), jnp.float32)
    zerosd = jnp.zeros((NPAD, DW), jnp.float32)

    w1 = jnp.stack([W1_0, W1_1, W1_2])
    w2 = jnp.stack([W2_0, W2_1, W2_2])
    b1 = _pad8(b1_0, b1_1, b1_2)
    b2 = _pad8(b2_0, b2_1, b2_2)
    bc8 = _pad8(bc)

    counts = _deg_kernel(idx6, onesd, zerosd)
    xt1 = _prep_call(x, counts)
    agg1 = _agg_kernel(xt1, sidx, didx, zeros128)
    xt2 = _dense1_call(agg1, counts, w1, b1)
    agg2 = _agg_kernel(xt2, sidx, didx, zeros128)
    out = _dense2_call(agg2, counts, w2, b2, Wc, bc8)
    return out


# speedup vs baseline: 1.2167x; 1.0202x over previous
import functools

import jax
import jax.numpy as jnp
from jax import lax
from jax.experimental import pallas as pl
from jax.experimental.pallas import tpu as pltpu
from jax.experimental.pallas import tpu_sc as plsc

N = 10000
D = 256
H = 256
C = 16
E = 100000
NREL = 3

NC = 2
NS = 16
K = 125
NCH = E // (NS * K)
NPAD = 10240
RPT = NPAD // NS
HD = H // NC

_mesh = plsc.VectorSubcoreMesh(core_axis_name="c", subcore_axis_name="s")


DW = 128


@functools.partial(
    pl.kernel,
    out_type=jax.ShapeDtypeStruct((6, NPAD, DW), jnp.float32),
    mesh=_mesh,
    scratch_types=[
        pltpu.VMEM((K, DW), jnp.float32),
        pltpu.VMEM((NCH, K), jnp.int32),
        pltpu.MemorySpace.VMEM_SHARED((NPAD, DW), jnp.float32),
        pltpu.SemaphoreType.DMA,
    ],
)
def _deg_kernel(idx_hbm, ones_hbm, zeros_hbm, out_hbm, ones_v, idx_v, acc, ssem):
    c = lax.axis_index("c")
    s = lax.axis_index("s")
    pltpu.sync_copy(ones_hbm, ones_v)

    def _wait_s():
        pltpu.make_async_copy(ones_v, acc.at[idx_v.at[0]], ssem).wait()

    for a in range(3):
        g = 3 * c + a
        pltpu.sync_copy(zeros_hbm.at[pl.ds(s * RPT, RPT)],
                        acc.at[pl.ds(s * RPT, RPT)])
        pltpu.sync_copy(idx_hbm.at[g, s], idx_v)
        plsc.subcore_barrier()

        @pl.loop(0, NCH)
        def _(j):
            pltpu.async_copy(ones_v, acc.at[idx_v.at[j]], ssem, add=True)

            @pl.when(j >= 3)
            def _():
                _wait_s()

        for _ in range(3):
            _wait_s()
        plsc.subcore_barrier()
        pltpu.sync_copy(acc.at[pl.ds(s * RPT, RPT)],
                        out_hbm.at[g, pl.ds(s * RPT, RPT)])
        plsc.subcore_barrier()


@functools.partial(
    pl.kernel,
    out_type=jax.ShapeDtypeStruct((NREL, NC, NPAD, HD), jnp.float32),
    mesh=_mesh,
    scratch_types=[
        pltpu.VMEM((NCH, K), jnp.int32),
        pltpu.VMEM((NCH, K), jnp.int32),
        pltpu.VMEM((K, HD), jnp.float32),
        pltpu.VMEM((K, HD), jnp.float32),
        pltpu.MemorySpace.VMEM_SHARED((NPAD, HD), jnp.float32),
        pltpu.SemaphoreType.DMA,
        pltpu.SemaphoreType.DMA,
        pltpu.SemaphoreType.DMA,
        pltpu.SemaphoreType.DMA,
    ],
)
def _agg_kernel(xt_hbm, sidx_hbm, didx_hbm, zeros_hbm, out_hbm,
                src_v, dst_v, buf0, buf1, acc, gs0, gs1, ss0, ss1):
    c = lax.axis_index("c")
    s = lax.axis_index("s")
    for r in range(NREL):
        pltpu.sync_copy(zeros_hbm.at[pl.ds(s * RPT, RPT)],
                        acc.at[pl.ds(s * RPT, RPT)])
        pltpu.sync_copy(sidx_hbm.at[r, s], src_v)
        pltpu.sync_copy(didx_hbm.at[r, s], dst_v)
        plsc.subcore_barrier()

        table = xt_hbm.at[r, c]
        pltpu.async_copy(table.at[src_v.at[0]], buf0, gs0)

        @pl.loop(0, NCH // 2)
        def _(i):
            j0 = 2 * i
            j1 = 2 * i + 1
            pltpu.async_copy(table.at[src_v.at[j1]], buf1, gs1)
            pltpu.make_async_copy(table.at[src_v.at[0]], buf0, gs0).wait()
            pltpu.sync_copy(buf0, acc.at[dst_v.at[j0]], add=True)

            @pl.when(j1 + 1 < NCH)
            def _():
                pltpu.async_copy(table.at[src_v.at[j1 + 1]], buf0, gs0)

            pltpu.make_async_copy(table.at[src_v.at[0]], buf1, gs1).wait()
            pltpu.sync_copy(buf1, acc.at[dst_v.at[j1]], add=True)

        plsc.subcore_barrier()
        pltpu.sync_copy(acc.at[pl.ds(s * RPT, RPT)],
                        out_hbm.at[r, c, pl.ds(s * RPT, RPT)])
        plsc.subcore_barrier()


_BR = 1024
_NB = NPAD // _BR


def _prep_body(x_ref, cnt_ref, xt_ref, dinv_ref):
    cnt = cnt_ref[...][:, :, 0]
    dinvs = lax.rsqrt(jnp.maximum(cnt, 1.0))
    dinv_ref[...] = jnp.concatenate(
        [dinvs, jnp.zeros((2, _BR), jnp.float32)], axis=0)[None]
    xb = x_ref[...]
    outs = []
    for r in range(NREL):
        xs = xb * dinvs[2 * r][:, None]
        outs.append(jnp.stack([xs[:, :HD], xs[:, HD:]], axis=0))
    xt_ref[...] = jnp.stack(outs, axis=0)


def _dense1_body(agg_ref, dinv_ref, w_ref, b_ref, xt_ref):
    dv = dinv_ref[...][0]
    agg = agg_ref[...]
    acc = jnp.zeros((_BR, H), jnp.float32)
    for r in range(NREL):
        a = jnp.concatenate([agg[r, 0], agg[r, 1]], axis=1)
        a = a * dv[2 * r + 1][:, None]
        acc = acc + jnp.dot(a, w_ref[...][r], preferred_element_type=jnp.float32)
    h = jnp.maximum(acc + jnp.sum(b_ref[...], axis=0)[None, :], 0.0)
    outs = []
    for r in range(NREL):
        hs = h * dv[2 * r][:, None]
        outs.append(jnp.stack([hs[:, :HD], hs[:, HD:]], axis=0))
    xt_ref[...] = jnp.stack(outs, axis=0)


def _dense2_body(agg_ref, dinv_ref, w_ref, b_ref, wc_ref, bc_ref, out_ref, acc_ref):
    dv = dinv_ref[...][0]
    agg = agg_ref[...]
    acc = jnp.zeros((_BR, H), jnp.float32)
    for r in range(NREL):
        a = jnp.concatenate([agg[r, 0], agg[r, 1]], axis=1)
        a = a * dv[2 * r + 1][:, None]
        acc = acc + jnp.dot(a, w_ref[...][r], preferred_element_type=jnp.float32)
    h2 = jnp.maximum(acc + jnp.sum(b_ref[...], axis=0)[None, :], 0.0)
    row = pl.program_id(0) * _BR + lax.broadcasted_iota(jnp.int32, (_BR, 1), 0)
    h2 = jnp.where(row < N, h2, 0.0)
    part = jnp.dot(jnp.ones((8, _BR), jnp.float32), h2,
                   preferred_element_type=jnp.float32)

    @pl.when(pl.program_id(0) == 0)
    def _():
        acc_ref[...] = jnp.zeros((8, H), jnp.float32)

    acc_ref[...] += part
    hg = acc_ref[0:1, :] * (1.0 / N)
    out_ref[...] = jnp.dot(hg, wc_ref[...], preferred_element_type=jnp.float32) \
        + bc_ref[0:1, :]


def _cnt_spec():
    return pl.BlockSpec((6, _BR, DW), lambda b: (0, b, 0))


def _dinv_spec():
    return pl.BlockSpec((1, 8, _BR), lambda b: (b, 0, 0))


def _agg_spec():
    return pl.BlockSpec((NREL, NC, _BR, HD), lambda b: (0, 0, b, 0))


def _xt_spec():
    return pl.BlockSpec((NREL, NC, _BR, HD), lambda b: (0, 0, b, 0))


_prep_call = pl.pallas_call(
    _prep_body,
    grid=(_NB,),
    in_specs=[pl.BlockSpec((_BR, D), lambda b: (b, 0)), _cnt_spec()],
    out_specs=(_xt_spec(), _dinv_spec()),
    out_shape=(jax.ShapeDtypeStruct((NREL, NC, N, HD), jnp.float32),
               jax.ShapeDtypeStruct((_NB, 8, _BR), jnp.float32)),
)

_dense1_call = pl.pallas_call(
    _dense1_body,
    grid=(_NB,),
    in_specs=[
        _agg_spec(),
        _dinv_spec(),
        pl.BlockSpec((NREL, H, H), lambda b: (0, 0, 0)),
        pl.BlockSpec((8, H), lambda b: (0, 0)),
    ],
    out_specs=_xt_spec(),
    out_shape=jax.ShapeDtypeStruct((NREL, NC, N, HD), jnp.float32),
)

_dense2_call = pl.pallas_call(
    _dense2_body,
    grid=(_NB,),
    in_specs=[
        _agg_spec(),
        _dinv_spec(),
        pl.BlockSpec((NREL, H, H), lambda b: (0, 0, 0)),
        pl.BlockSpec((8, H), lambda b: (0, 0)),
        pl.BlockSpec((H, C), lambda b: (0, 0)),
        pl.BlockSpec((8, C), lambda b: (0, 0)),
    ],
    out_specs=pl.BlockSpec((1, C), lambda b: (0, 0)),
    out_shape=jax.ShapeDtypeStruct((1, C), jnp.float32),
    scratch_shapes=[pltpu.VMEM((8, H), jnp.float32)],
)


def _pad8(*rows):
    z = jnp.zeros((8, rows[0].shape[0]), jnp.float32)
    for i, r in enumerate(rows):
        z = z.at[i].set(r)
    return z


def kernel(x, e0, e1, e2, W1_0, b1_0, W1_1, b1_1, W1_2, b1_2,
           W2_0, b2_0, W2_1, b2_1, W2_2, b2_2, Wc, bc):
    src = jnp.stack([e0[0], e1[0], e2[0]]).astype(jnp.int32)
    dst = jnp.stack([e0[1], e1[1], e2[1]]).astype(jnp.int32)
    sidx = src.reshape(NREL, NS, NCH, K)
    didx = dst.reshape(NREL, NS, NCH, K)
    idx6 = jnp.stack([src, dst], axis=1).reshape(6, NS, NCH, K)

    zeros128 = jnp.zeros((NPAD, HD), jnp.float32)
    onesd = jnp.ones((K, DW), jnp.float32)
    zerosd = jnp.zeros((NPAD, DW), jnp.float32)

    w1 = jnp.stack([W1_0, W1_1, W1_2])
    w2 = jnp.stack([W2_0, W2_1, W2_2])
    b1 = _pad8(b1_0, b1_1, b1_2)
    b2 = _pad8(b2_0, b2_1, b2_2)
    bc8 = _pad8(bc)

    counts = _deg_kernel(idx6, onesd, zerosd)
    xt1, dinv8 = _prep_call(x, counts)
    agg1 = _agg_kernel(xt1, sidx, didx, zeros128)
    xt2 = _dense1_call(agg1, dinv8, w1, b1)
    agg2 = _agg_kernel(xt2, sidx, didx, zeros128)
    out = _dense2_call(agg2, dinv8, w2, b2, Wc, bc8)
    return out


# speedup vs baseline: 1.2248x; 1.0066x over previous
import functools

import jax
import jax.numpy as jnp
from jax import lax
from jax.experimental import pallas as pl
from jax.experimental.pallas import tpu as pltpu
from jax.experimental.pallas import tpu_sc as plsc

N = 10000
D = 256
H = 256
C = 16
E = 100000
NREL = 3

NC = 2
NS = 16
K = 125
NCH = E // (NS * K)
NPAD = 10240
RPT = NPAD // NS
HD = H // NC

_mesh = plsc.VectorSubcoreMesh(core_axis_name="c", subcore_axis_name="s")


DW = 128


@functools.partial(
    pl.kernel,
    out_type=jax.ShapeDtypeStruct((6, NPAD, DW), jnp.float32),
    mesh=_mesh,
    scratch_types=[
        pltpu.VMEM((K, DW), jnp.float32),
        pltpu.VMEM((NCH, K), jnp.int32),
        pltpu.MemorySpace.VMEM_SHARED((NPAD, DW), jnp.float32),
        pltpu.SemaphoreType.DMA,
    ],
)
def _deg_kernel(idx_hbm, ones_hbm, zeros_hbm, out_hbm, ones_v, idx_v, acc, ssem):
    c = lax.axis_index("c")
    s = lax.axis_index("s")
    pltpu.sync_copy(ones_hbm, ones_v)

    def _wait_s():
        pltpu.make_async_copy(ones_v, acc.at[idx_v.at[0]], ssem).wait()

    for a in range(3):
        g = 3 * c + a
        pltpu.sync_copy(zeros_hbm.at[pl.ds(s * RPT, RPT)],
                        acc.at[pl.ds(s * RPT, RPT)])
        pltpu.sync_copy(idx_hbm.at[g, s], idx_v)
        plsc.subcore_barrier()

        @pl.loop(0, NCH)
        def _(j):
            pltpu.async_copy(ones_v, acc.at[idx_v.at[j]], ssem, add=True)

            @pl.when(j >= 3)
            def _():
                _wait_s()

        for _ in range(3):
            _wait_s()
        plsc.subcore_barrier()
        pltpu.sync_copy(acc.at[pl.ds(s * RPT, RPT)],
                        out_hbm.at[g, pl.ds(s * RPT, RPT)])
        plsc.subcore_barrier()


@functools.partial(
    pl.kernel,
    out_type=jax.ShapeDtypeStruct((NREL, NC, NPAD, HD), jnp.float32),
    mesh=_mesh,
    scratch_types=[
        pltpu.VMEM((NCH, K), jnp.int32),
        pltpu.VMEM((NCH, K), jnp.int32),
        pltpu.VMEM((K, HD), jnp.float32),
        pltpu.VMEM((K, HD), jnp.float32),
        pltpu.MemorySpace.VMEM_SHARED((NPAD, HD), jnp.float32),
        pltpu.SemaphoreType.DMA,
        pltpu.SemaphoreType.DMA,
        pltpu.SemaphoreType.DMA,
        pltpu.SemaphoreType.DMA,
    ],
)
def _agg_kernel(xt_hbm, sidx_hbm, didx_hbm, zeros_hbm, out_hbm,
                src_v, dst_v, buf0, buf1, acc, gs0, gs1, ss0, ss1):
    c = lax.axis_index("c")
    s = lax.axis_index("s")
    for r in range(NREL):
        pltpu.sync_copy(zeros_hbm.at[pl.ds(s * RPT, RPT)],
                        acc.at[pl.ds(s * RPT, RPT)])
        pltpu.sync_copy(sidx_hbm.at[r, s], src_v)
        pltpu.sync_copy(didx_hbm.at[r, s], dst_v)
        plsc.subcore_barrier()

        table = xt_hbm.at[r, c]
        pltpu.async_copy(table.at[src_v.at[0]], buf0, gs0)

        @pl.loop(0, NCH // 2)
        def _(i):
            j0 = 2 * i
            j1 = 2 * i + 1
            pltpu.async_copy(table.at[src_v.at[j1]], buf1, gs1)
            pltpu.make_async_copy(table.at[src_v.at[0]], buf0, gs0).wait()
            pltpu.sync_copy(buf0, acc.at[dst_v.at[j0]], add=True)

            @pl.when(j1 + 1 < NCH)
            def _():
                pltpu.async_copy(table.at[src_v.at[j1 + 1]], buf0, gs0)

            pltpu.make_async_copy(table.at[src_v.at[0]], buf1, gs1).wait()
            pltpu.sync_copy(buf1, acc.at[dst_v.at[j1]], add=True)

        plsc.subcore_barrier()
        pltpu.sync_copy(acc.at[pl.ds(s * RPT, RPT)],
                        out_hbm.at[r, c, pl.ds(s * RPT, RPT)])
        plsc.subcore_barrier()


_BR = 2048
_NB = NPAD // _BR


def _prep_body(x_ref, cnt_ref, xt_ref, dinv_ref):
    cnt = cnt_ref[...][:, :, 0]
    dinvs = lax.rsqrt(jnp.maximum(cnt, 1.0))
    dinv_ref[...] = jnp.concatenate(
        [dinvs, jnp.zeros((2, _BR), jnp.float32)], axis=0)[None]
    xb = x_ref[...]
    outs = []
    for r in range(NREL):
        xs = xb * dinvs[2 * r][:, None]
        outs.append(jnp.stack([xs[:, :HD], xs[:, HD:]], axis=0))
    xt_ref[...] = jnp.stack(outs, axis=0)


def _dense1_body(agg_ref, dinv_ref, w_ref, b_ref, xt_ref):
    dv = dinv_ref[...][0]
    agg = agg_ref[...]
    acc = jnp.zeros((_BR, H), jnp.float32)
    for r in range(NREL):
        a = jnp.concatenate([agg[r, 0], agg[r, 1]], axis=1)
        a = a * dv[2 * r + 1][:, None]
        acc = acc + jnp.dot(a, w_ref[...][r], preferred_element_type=jnp.float32)
    h = jnp.maximum(acc + jnp.sum(b_ref[...], axis=0)[None, :], 0.0)
    outs = []
    for r in range(NREL):
        hs = h * dv[2 * r][:, None]
        outs.append(jnp.stack([hs[:, :HD], hs[:, HD:]], axis=0))
    xt_ref[...] = jnp.stack(outs, axis=0)


def _dense2_body(agg_ref, dinv_ref, w_ref, b_ref, wc_ref, bc_ref, out_ref, acc_ref):
    dv = dinv_ref[...][0]
    agg = agg_ref[...]
    acc = jnp.zeros((_BR, H), jnp.float32)
    for r in range(NREL):
        a = jnp.concatenate([agg[r, 0], agg[r, 1]], axis=1)
        a = a * dv[2 * r + 1][:, None]
        acc = acc + jnp.dot(a, w_ref[...][r], preferred_element_type=jnp.float32)
    h2 = jnp.maximum(acc + jnp.sum(b_ref[...], axis=0)[None, :], 0.0)
    row = pl.program_id(0) * _BR + lax.broadcasted_iota(jnp.int32, (_BR, 1), 0)
    h2 = jnp.where(row < N, h2, 0.0)
    part = jnp.dot(jnp.ones((8, _BR), jnp.float32), h2,
                   preferred_element_type=jnp.float32)

    @pl.when(pl.program_id(0) == 0)
    def _():
        acc_ref[...] = jnp.zeros((8, H), jnp.float32)

    acc_ref[...] += part
    hg = acc_ref[0:1, :] * (1.0 / N)
    out_ref[...] = jnp.dot(hg, wc_ref[...], preferred_element_type=jnp.float32) \
        + bc_ref[0:1, :]


def _cnt_spec():
    return pl.BlockSpec((6, _BR, DW), lambda b: (0, b, 0))


def _dinv_spec():
    return pl.BlockSpec((1, 8, _BR), lambda b: (b, 0, 0))


def _agg_spec():
    return pl.BlockSpec((NREL, NC, _BR, HD), lambda b: (0, 0, b, 0))


def _xt_spec():
    return pl.BlockSpec((NREL, NC, _BR, HD), lambda b: (0, 0, b, 0))


_prep_call = pl.pallas_call(
    _prep_body,
    grid=(_NB,),
    in_specs=[pl.BlockSpec((_BR, D), lambda b: (b, 0)), _cnt_spec()],
    out_specs=(_xt_spec(), _dinv_spec()),
    out_shape=(jax.ShapeDtypeStruct((NREL, NC, N, HD), jnp.float32),
               jax.ShapeDtypeStruct((_NB, 8, _BR), jnp.float32)),
)

_dense1_call = pl.pallas_call(
    _dense1_body,
    grid=(_NB,),
    in_specs=[
        _agg_spec(),
        _dinv_spec(),
        pl.BlockSpec((NREL, H, H), lambda b: (0, 0, 0)),
        pl.BlockSpec((8, H), lambda b: (0, 0)),
    ],
    out_specs=_xt_spec(),
    out_shape=jax.ShapeDtypeStruct((NREL, NC, N, HD), jnp.float32),
)

_dense2_call = pl.pallas_call(
    _dense2_body,
    grid=(_NB,),
    in_specs=[
        _agg_spec(),
        _dinv_spec(),
        pl.BlockSpec((NREL, H, H), lambda b: (0, 0, 0)),
        pl.BlockSpec((8, H), lambda b: (0, 0)),
        pl.BlockSpec((H, C), lambda b: (0, 0)),
        pl.BlockSpec((8, C), lambda b: (0, 0)),
    ],
    out_specs=pl.BlockSpec((1, C), lambda b: (0, 0)),
    out_shape=jax.ShapeDtypeStruct((1, C), jnp.float32),
    scratch_shapes=[pltpu.VMEM((8, H), jnp.float32)],
)


def _pad8(*rows):
    z = jnp.zeros((8, rows[0].shape[0]), jnp.float32)
    for i, r in enumerate(rows):
        z = z.at[i].set(r)
    return z


def kernel(x, e0, e1, e2, W1_0, b1_0, W1_1, b1_1, W1_2, b1_2,
           W2_0, b2_0, W2_1, b2_1, W2_2, b2_2, Wc, bc):
    src = jnp.stack([e0[0], e1[0], e2[0]]).astype(jnp.int32)
    dst = jnp.stack([e0[1], e1[1], e2[1]]).astype(jnp.int32)
    sidx = src.reshape(NREL, NS, NCH, K)
    didx = dst.reshape(NREL, NS, NCH, K)
    idx6 = jnp.stack([src, dst], axis=1).reshape(6, NS, NCH, K)

    zeros128 = jnp.zeros((NPAD, HD), jnp.float32)
    onesd = jnp.ones((K, DW), jnp.float32)
    zerosd = jnp.zeros((NPAD, DW), jnp.float32)

    w1 = jnp.stack([W1_0, W1_1, W1_2])
    w2 = jnp.stack([W2_0, W2_1, W2_2])
    b1 = _pad8(b1_0, b1_1, b1_2)
    b2 = _pad8(b2_0, b2_1, b2_2)
    bc8 = _pad8(bc)

    counts = _deg_kernel(idx6, onesd, zerosd)
    xt1, dinv8 = _prep_call(x, counts)
    agg1 = _agg_kernel(xt1, sidx, didx, zeros128)
    xt2 = _dense1_call(agg1, dinv8, w1, b1)
    agg2 = _agg_kernel(xt2, sidx, didx, zeros128)
    out = _dense2_call(agg2, dinv8, w2, b2, Wc, bc8)
    return out
